# Initial kernel scaffold; baseline (speedup 1.0000x reference)
import jax
import jax.numpy as jnp
from jax.experimental import pallas as pl


def kernel(src, dst, edge_index, W, b):
    raise NotImplementedError("write your pallas kernel here")


# speedup vs baseline: 3.5389x; 3.5389x over previous
import functools

import jax
import jax.numpy as jnp
from jax import lax
from jax.experimental import pallas as pl
from jax.experimental.pallas import tpu as pltpu
from jax.experimental.pallas import tpu_sc as plsc

N_NODES = 10000
N_EDGES = 160000
DIM = 256

NC = 2
NS = 16
NW = NC * NS
TILE_E = N_EDGES // NW
PAD_TILE = 5008
E_PAD = NW * PAD_TILE
NVREG = PAD_TILE // 16
CH = 40
NCHUNK = TILE_E // CH

_mesh = plsc.VectorSubcoreMesh(core_axis_name="c", subcore_axis_name="s")
_sc_params = pltpu.CompilerParams(needs_layout_passes=False)


def _node_scores_body(src_ref, dst_ref, w1_ref, w2_ref, ssrc_ref, sdst_ref):
    ssrc_ref[...] = jnp.sum(src_ref[...] * w1_ref[...], axis=1)
    sdst_ref[...] = jnp.sum(dst_ref[...] * w2_ref[...], axis=1)


_node_scores = pl.pallas_call(
    _node_scores_body,
    out_shape=[
        jax.ShapeDtypeStruct((N_NODES,), jnp.float32),
        jax.ShapeDtypeStruct((N_NODES,), jnp.float32),
    ],
)


@functools.partial(
    pl.kernel,
    mesh=_mesh,
    compiler_params=_sc_params,
    out_type=jax.ShapeDtypeStruct((E_PAD,), jnp.float32),
    scratch_types=[
        pltpu.VMEM((N_NODES,), jnp.float32),
        pltpu.VMEM((N_NODES,), jnp.float32),
        pltpu.VMEM((PAD_TILE,), jnp.int32),
        pltpu.VMEM((PAD_TILE,), jnp.int32),
        pltpu.VMEM((PAD_TILE,), jnp.float32),
    ],
)
def _edge_scores(ssrc_hbm, sdst_hbm, esrc_hbm, edst_hbm, out_hbm,
                 ssrc_v, sdst_v, esrc_v, edst_v, sc_v):
    wid = lax.axis_index("s") * NC + lax.axis_index("c")
    pbase = wid * PAD_TILE
    pltpu.sync_copy(ssrc_hbm, ssrc_v)
    pltpu.sync_copy(sdst_hbm, sdst_v)
    pltpu.sync_copy(esrc_hbm.at[pl.ds(pbase, PAD_TILE)], esrc_v)
    pltpu.sync_copy(edst_hbm.at[pl.ds(pbase, PAD_TILE)], edst_v)

    def body(i, carry):
        sl = pl.ds(i * 16, 16)
        a = plsc.load_gather(ssrc_v, [esrc_v[sl]])
        d = plsc.load_gather(sdst_v, [edst_v[sl]])
        sc_v[sl] = a + d
        return carry

    lax.fori_loop(0, NVREG, body, 0)
    pltpu.sync_copy(sc_v, out_hbm.at[pl.ds(pbase, PAD_TILE)])


def _softmax_body(sc_ref, out_ref):
    x = sc_ref[...]
    r = lax.broadcasted_iota(jnp.int32, x.shape, 0)
    c = lax.broadcasted_iota(jnp.int32, x.shape, 1)
    flat = r * 128 + c
    valid = (flat % PAD_TILE) < TILE_E
    xm = jnp.where(valid, x, jnp.float32(-1e30))
    m = jnp.max(xm)
    e = jnp.where(valid, jnp.exp(x - m), jnp.float32(0.0))
    s = jnp.sum(e)
    out_ref[...] = e * (1.0 / s)


_softmax = pl.pallas_call(
    _softmax_body,
    out_shape=jax.ShapeDtypeStruct((E_PAD // 128, 128), jnp.float32),
)


@functools.partial(
    pl.kernel,
    mesh=_mesh,
    compiler_params=_sc_params,
    out_type=jax.ShapeDtypeStruct((N_EDGES, DIM), jnp.float32),
    scratch_types=[
        pltpu.VMEM((TILE_E,), jnp.int32),
        pltpu.VMEM((PAD_TILE,), jnp.float32),
        pltpu.VMEM((CH, DIM), jnp.float32),
        pltpu.SemaphoreType.DMA,
    ],
)
def _gather_scale(src_hbm, esrc_hbm, w_hbm, out_hbm, idx_v, wt_v, rows_v, sem):
    wid = lax.axis_index("s") * NC + lax.axis_index("c")
    ebase = wid * TILE_E
    pbase = wid * PAD_TILE
    pltpu.sync_copy(esrc_hbm.at[pl.ds(pbase, TILE_E)], idx_v)
    pltpu.sync_copy(w_hbm.at[pl.ds(pbase, PAD_TILE)], wt_v)

    def chunk(i, carry):
        off = i * CH
        pltpu.async_copy(src_hbm.at[idx_v.at[pl.ds(off, CH)]], rows_v, sem).wait()

        wvecs = [wt_v[pl.ds(off + g * 16, 16)] for g in range((CH + 15) // 16)]
        for e in range(CH):
            w = wvecs[e // 16][e % 16]
            for j in range(DIM // 16):
                sl = pl.ds(j * 16, 16)
                rows_v[e, sl] = rows_v[e, sl] * w

        pltpu.sync_copy(rows_v, out_hbm.at[pl.ds(ebase + off, CH)])
        return carry

    lax.fori_loop(0, NCHUNK, chunk, 0)


@jax.jit
def kernel(src, dst, edge_index, W, b):
    del b
    edge_index = edge_index.astype(jnp.int32)
    w1 = W[:DIM, 0].reshape(1, DIM)
    w2 = W[DIM:, 0].reshape(1, DIM)
    s_src, s_dst = _node_scores(src, dst, w1, w2)

    pad = ((0, 0), (0, PAD_TILE - TILE_E))
    esrc_p = jnp.pad(edge_index[0].reshape(NW, TILE_E), pad).reshape(-1)
    edst_p = jnp.pad(edge_index[1].reshape(NW, TILE_E), pad).reshape(-1)

    scores_p = _edge_scores(s_src, s_dst, esrc_p, edst_p)
    weights_p = _softmax(scores_p.reshape(E_PAD // 128, 128)).reshape(-1)
    return _gather_scale(src, esrc_p, weights_p)


# speedup vs baseline: 5.8261x; 1.6463x over previous
import functools

import jax
import jax.numpy as jnp
from jax import lax
from jax.experimental import pallas as pl
from jax.experimental.pallas import tpu as pltpu
from jax.experimental.pallas import tpu_sc as plsc

N_NODES = 10000
N_EDGES = 160000
DIM = 256

NC = 2
NS = 16
NW = NC * NS
TILE_E = N_EDGES // NW
NVREG = TILE_E // 16
IDX_PAD = TILE_E + 16
CH = 40
NCHUNK = TILE_E // CH
NBUF = 5
NROUND = NCHUNK // NBUF

_mesh = plsc.VectorSubcoreMesh(core_axis_name="c", subcore_axis_name="s")
_sc_params = pltpu.CompilerParams(needs_layout_passes=False)


def _node_scores_body(src_ref, dst_ref, w1_ref, w2_ref, ssrc_ref, sdst_ref):
    ssrc_ref[...] = jnp.sum(src_ref[...] * w1_ref[...], axis=1)
    sdst_ref[...] = jnp.sum(dst_ref[...] * w2_ref[...], axis=1)


_node_scores = pl.pallas_call(
    _node_scores_body,
    out_shape=[
        jax.ShapeDtypeStruct((N_NODES,), jnp.float32),
        jax.ShapeDtypeStruct((N_NODES,), jnp.float32),
    ],
)


@functools.partial(
    pl.kernel,
    mesh=_mesh,
    compiler_params=_sc_params,
    out_type=jax.ShapeDtypeStruct((N_EDGES,), jnp.float32),
    scratch_types=[
        pltpu.VMEM((N_NODES,), jnp.float32),
        pltpu.VMEM((N_NODES,), jnp.float32),
        pltpu.VMEM((IDX_PAD,), jnp.int32),
        pltpu.VMEM((IDX_PAD,), jnp.int32),
        pltpu.VMEM((IDX_PAD,), jnp.float32),
    ],
)
def _edge_scores(ssrc_hbm, sdst_hbm, esrc_hbm, edst_hbm, out_hbm,
                 ssrc_v, sdst_v, esrc_v, edst_v, sc_v):
    wid = lax.axis_index("s") * NC + lax.axis_index("c")
    base = wid * TILE_E
    pltpu.sync_copy(ssrc_hbm, ssrc_v)
    pltpu.sync_copy(sdst_hbm, sdst_v)
    pltpu.sync_copy(esrc_hbm.at[pl.ds(base, TILE_E)],
                    esrc_v.at[pl.ds(0, TILE_E)])
    pltpu.sync_copy(edst_hbm.at[pl.ds(base, TILE_E)],
                    edst_v.at[pl.ds(0, TILE_E)])

    def body(i, carry):
        sl = pl.ds(i * 16, 16)
        a = plsc.load_gather(ssrc_v, [esrc_v[sl]])
        d = plsc.load_gather(sdst_v, [edst_v[sl]])
        sc_v[sl] = a + d
        return carry

    lax.fori_loop(0, NVREG, body, 0)

    tl = pl.ds(NVREG * 16, 16)
    mask = lax.iota(jnp.int32, 16) < (TILE_E - NVREG * 16)
    i_s = jnp.where(mask, esrc_v[tl], 0)
    i_d = jnp.where(mask, edst_v[tl], 0)
    sc_v[tl] = plsc.load_gather(ssrc_v, [i_s]) + plsc.load_gather(sdst_v, [i_d])

    pltpu.sync_copy(sc_v.at[pl.ds(0, TILE_E)], out_hbm.at[pl.ds(base, TILE_E)])


def _softmax_body(sc_ref, out_ref):
    x = sc_ref[...]
    m = jnp.max(x)
    e = jnp.exp(x - m)
    out_ref[...] = e * (1.0 / jnp.sum(e))


_softmax = pl.pallas_call(
    _softmax_body,
    out_shape=jax.ShapeDtypeStruct((N_EDGES // 128, 128), jnp.float32),
)


@functools.partial(
    pl.kernel,
    mesh=_mesh,
    compiler_params=_sc_params,
    out_type=jax.ShapeDtypeStruct((N_EDGES, DIM), jnp.float32),
    scratch_types=[
        pltpu.VMEM((TILE_E,), jnp.int32),
        pltpu.VMEM((TILE_E,), jnp.float32),
        *[pltpu.VMEM((CH, DIM), jnp.float32) for _ in range(NBUF)],
        *[pltpu.SemaphoreType.DMA for _ in range(2 * NBUF)],
    ],
)
def _gather_scale(src_hbm, esrc_hbm, w_hbm, out_hbm, idx_v, wt_v, *bufs_sems):
    rows = bufs_sems[:NBUF]
    gsem = bufs_sems[NBUF:2 * NBUF]
    osem = bufs_sems[2 * NBUF:]
    wid = lax.axis_index("s") * NC + lax.axis_index("c")
    base = wid * TILE_E
    pltpu.sync_copy(esrc_hbm.at[pl.ds(base, TILE_E)], idx_v)
    pltpu.sync_copy(w_hbm.at[pl.ds(base, TILE_E)], wt_v)

    def gather(ci, b):
        return pltpu.make_async_copy(
            src_hbm.at[idx_v.at[pl.ds(ci * CH, CH)]], rows[b], gsem[b])

    def writeout(ci, b):
        return pltpu.make_async_copy(
            rows[b], out_hbm.at[pl.ds(base + ci * CH, CH)], osem[b])

    def scale(off, b):
        rb = rows[b]

        def srow(e, c):
            w = plsc.load_gather(wt_v, [jnp.full((16,), off + e, jnp.int32)])
            for j in range(DIM // 16):
                sl = pl.ds(j * 16, 16)
                rb[e, sl] = rb[e, sl] * w
            return c

        lax.fori_loop(0, CH, srow, 0)

    for b in range(NBUF):
        gather(b, b).start()

    def round_body(g, carry, issue_next):
        for b in range(NBUF):
            ci = g * NBUF + b
            gather(ci, b).wait()
            scale(ci * CH, b)
            writeout(ci, b).start()
        if issue_next:
            for b in range(NBUF):
                ci = g * NBUF + b
                writeout(ci, b).wait()
                gather(ci + NBUF, b).start()
        return carry

    lax.fori_loop(0, NROUND - 1, lambda g, c: round_body(g, c, True), 0)
    round_body(NROUND - 1, 0, False)
    for b in range(NBUF):
        writeout((NROUND - 1) * NBUF + b, b).wait()


@jax.jit
def kernel(src, dst, edge_index, W, b):
    del b
    edge_index = edge_index.astype(jnp.int32)
    w1 = W[:DIM, 0].reshape(1, DIM)
    w2 = W[DIM:, 0].reshape(1, DIM)
    s_src, s_dst = _node_scores(src, dst, w1, w2)
    scores = _edge_scores(s_src, s_dst, edge_index[0], edge_index[1])
    weights = _softmax(scores.reshape(N_EDGES // 128, 128)).reshape(-1)
    return _gather_scale(src, edge_index[0], weights)
